# baseline (speedup 1.0000x reference)
import functools

import jax
import jax.numpy as jnp
from jax import lax
from jax.experimental import pallas as pl
from jax.experimental.pallas import tpu as pltpu
from jax.experimental.pallas import tpu_sc as plsc

VOCAB = 1000000
EMBED = 64
BATCH = 16384
PAIR = 2 * EMBED

NUM_CORES = 2
NUM_SUBCORES = 16
LANES = 16
NUM_WORKERS = NUM_CORES * NUM_SUBCORES
ROWS_PER_WORKER = BATCH // NUM_WORKERS
HALF = ROWS_PER_WORKER // 2
CHUNK = 128
GROUPS = HALF // LANES


def _sc_body(x_hbm, t_hbm, table_hbm, out_hbm,
             idx_x, idx_t, pidx_x, pidx_t, rows_x, rows_t, out_v, sem):
    wid = lax.axis_index("s") * NUM_CORES + lax.axis_index("c")
    base = wid * ROWS_PER_WORKER

    pltpu.sync_copy(x_hbm.at[pl.ds(base, ROWS_PER_WORKER)], idx_x)
    pltpu.sync_copy(t_hbm.at[pl.ds(base, ROWS_PER_WORKER)], idx_t)

    def shift(i, carry):
        pidx_x[pl.ds(i * LANES, LANES)] = idx_x[pl.ds(i * LANES, LANES)] >> 1
        pidx_t[pl.ds(i * LANES, LANES)] = idx_t[pl.ds(i * LANES, LANES)] >> 1
        return carry

    lax.fori_loop(0, ROWS_PER_WORKER // LANES, shift, 0)

    lanes = lax.iota(jnp.int32, LANES)

    def half(h, carry):
        hb = h * HALF
        copies = []
        for j in range(HALF // CHUNK):
            copies.append(pltpu.async_copy(
                table_hbm.at[pidx_x.at[pl.ds(hb + j * CHUNK, CHUNK)]],
                rows_x.at[pl.ds(j * CHUNK, CHUNK)], sem))
            copies.append(pltpu.async_copy(
                table_hbm.at[pidx_t.at[pl.ds(hb + j * CHUNK, CHUNK)]],
                rows_t.at[pl.ds(j * CHUNK, CHUNK)], sem))
        for c in copies:
            c.wait()

        def group(g, carry2):
            r = hb + g * LANES
            vx = idx_x[pl.ds(r, LANES)]
            vt = idx_t[pl.ds(r, LANES)]
            ridx = g * LANES + lanes
            cx = (vx & 1) * EMBED
            ct = (vt & 1) * EMBED
            acc = jnp.zeros((LANES,), jnp.float32)
            for d in range(EMBED):
                gx = plsc.load_gather(rows_x, [ridx, cx + d])
                gt = plsc.load_gather(rows_t, [ridx, ct + d])
                acc = acc + gx * gt
            out_v[pl.ds(r, LANES)] = acc
            return carry2

        lax.fori_loop(0, GROUPS, group, 0)
        return carry

    lax.fori_loop(0, 2, half, 0)

    pltpu.sync_copy(out_v, out_hbm.at[pl.ds(base, ROWS_PER_WORKER)])


@jax.jit
def kernel(x, t, table):
    mesh = plsc.VectorSubcoreMesh(core_axis_name="c", subcore_axis_name="s",
                                  num_cores=NUM_CORES,
                                  num_subcores=NUM_SUBCORES)
    run = pl.kernel(
        _sc_body,
        out_type=jax.ShapeDtypeStruct((BATCH,), jnp.float32),
        mesh=mesh,
        scratch_types=[
            pltpu.VMEM((ROWS_PER_WORKER,), jnp.int32),
            pltpu.VMEM((ROWS_PER_WORKER,), jnp.int32),
            pltpu.VMEM((ROWS_PER_WORKER,), jnp.int32),
            pltpu.VMEM((ROWS_PER_WORKER,), jnp.int32),
            pltpu.VMEM((HALF, PAIR), jnp.float32),
            pltpu.VMEM((HALF, PAIR), jnp.float32),
            pltpu.VMEM((ROWS_PER_WORKER,), jnp.float32),
            pltpu.SemaphoreType.DMA,
        ],
        compiler_params=pltpu.CompilerParams(needs_layout_passes=False,
                                             use_tc_tiling_on_sc=True),
    )
    return run(x, t, table.reshape(VOCAB // 2, PAIR))


# speedup vs baseline: 2.2324x; 2.2324x over previous
import functools

import jax
import jax.numpy as jnp
from jax import lax
from jax.experimental import pallas as pl
from jax.experimental.pallas import tpu as pltpu
from jax.experimental.pallas import tpu_sc as plsc

VOCAB = 1000000
EMBED = 64
BATCH = 16384

NUM_CORES = 2
NUM_SUBCORES = 16
LANES = 16
NUM_WORKERS = NUM_CORES * NUM_SUBCORES
ROWS_PER_WORKER = BATCH // NUM_WORKERS
HALF = ROWS_PER_WORKER // 2
K = 16
STEPS = HALF // K
GROUPS = HALF // LANES


def _sc_body(x_hbm, t_hbm, table_hbm, out_hbm,
             idx_x, idx_t, rows_x, rows_t, out_v, sem):
    wid = lax.axis_index("s") * NUM_CORES + lax.axis_index("c")
    base = wid * ROWS_PER_WORKER

    pltpu.sync_copy(x_hbm.at[pl.ds(base, ROWS_PER_WORKER)], idx_x)
    pltpu.sync_copy(t_hbm.at[pl.ds(base, ROWS_PER_WORKER)], idx_t)

    lanes = lax.iota(jnp.int32, LANES)

    def half(h, carry):
        hb = h * HALF

        def step(s, carry2):
            vecx = idx_x[pl.ds(hb + s * K, K)]
            vect = idx_t[pl.ds(hb + s * K, K)]
            copies = []
            for k in range(K):
                r = s * K + k
                copies.append(pltpu.async_copy(
                    table_hbm.at[0, pl.ds(vecx[k], 1)],
                    rows_x.at[pl.ds(r, 1)], sem))
                copies.append(pltpu.async_copy(
                    table_hbm.at[0, pl.ds(vect[k], 1)],
                    rows_t.at[pl.ds(r, 1)], sem))
            for c in copies:
                c.wait()
            return carry2

        lax.fori_loop(0, STEPS, step, 0)

        def group(g, carry2):
            ridx = g * LANES + lanes
            acc = jnp.zeros((LANES,), jnp.float32)
            for d in range(EMBED):
                gx = plsc.load_gather(rows_x, [ridx, jnp.full((LANES,), d, jnp.int32)])
                gt = plsc.load_gather(rows_t, [ridx, jnp.full((LANES,), d, jnp.int32)])
                acc = acc + gx * gt
            out_v[pl.ds(hb + g * LANES, LANES)] = acc
            return carry2

        lax.fori_loop(0, GROUPS, group, 0)
        return carry

    lax.fori_loop(0, 2, half, 0)

    pltpu.sync_copy(out_v, out_hbm.at[pl.ds(base, ROWS_PER_WORKER)])


@jax.jit
def kernel(x, t, table):
    mesh = plsc.VectorSubcoreMesh(core_axis_name="c", subcore_axis_name="s",
                                  num_cores=NUM_CORES,
                                  num_subcores=NUM_SUBCORES)
    run = pl.kernel(
        _sc_body,
        out_type=jax.ShapeDtypeStruct((BATCH,), jnp.float32),
        mesh=mesh,
        scratch_types=[
            pltpu.VMEM((ROWS_PER_WORKER,), jnp.int32),
            pltpu.VMEM((ROWS_PER_WORKER,), jnp.int32),
            pltpu.VMEM((HALF, EMBED), jnp.float32),
            pltpu.VMEM((HALF, EMBED), jnp.float32),
            pltpu.VMEM((ROWS_PER_WORKER,), jnp.float32),
            pltpu.SemaphoreType.DMA,
        ],
        compiler_params=pltpu.CompilerParams(needs_layout_passes=False,
                                             use_tc_tiling_on_sc=True),
    )
    return run(x, t, table.reshape(1, VOCAB, EMBED))
